# Initial kernel scaffold; baseline (speedup 1.0000x reference)
import jax
import jax.numpy as jnp
from jax.experimental import pallas as pl


def kernel(inputs, W_route, b_route, W_noise, b_noise, A_logs, noise_eps):
    raise NotImplementedError("write your pallas kernel here")


# speedup vs baseline: 1.7113x; 1.7113x over previous
import functools

import jax
import jax.numpy as jnp
from jax import lax
from jax.experimental import pallas as pl
from jax.experimental.pallas import tpu as pltpu

_NUM_EXPERTS = 16
_TOP_K = 2
_B = 32
_D_IN = 2048
_HW = 256
_DN = 8192 * 64
_N_COMBINE = 16
_A_BLK = _DN // _N_COMBINE


def _fused_kernel(x_ref, wr_ref, br_ref, wn_ref, bn_ref, eps_ref, a_ref,
                  comb_ref, z_ref, mh_ref, w_ref):
    i = pl.program_id(0)

    @pl.when(i < _B)
    def _reduce():
        x = x_ref[0]
        mh_ref[pl.ds(i, 1), :] = jnp.sum(x, axis=-1)[None, :] * (1.0 / _HW)

    @pl.when(i == _B)
    def _route():
        mh = mh_ref[...]
        lin_r = jnp.dot(mh, wr_ref[...], preferred_element_type=jnp.float32) + br_ref[...]
        p = jax.nn.softmax(lin_r, axis=-1)
        lin_n = jnp.dot(mh, wn_ref[...], preferred_element_type=jnp.float32) + bn_ref[...]
        s = eps_ref[...] * jax.nn.softplus(lin_n)
        q = jax.nn.softmax(s, axis=-1)
        noisy = p + q

        zl = jnp.log(jnp.sum(jnp.exp(noisy), axis=-1))
        z_ref[0, 0] = jnp.sum(zl * zl) * (1.0 / _B)

        iota = lax.broadcasted_iota(jnp.int32, (_B, _NUM_EXPERTS), 1)
        m1 = jnp.max(noisy, axis=1, keepdims=True)
        i1 = jnp.min(jnp.where(noisy == m1, iota, _NUM_EXPERTS), axis=1, keepdims=True)
        rest = jnp.where(iota == i1, -jnp.inf, noisy)
        m2 = jnp.max(rest, axis=1, keepdims=True)
        i2 = jnp.min(jnp.where(rest == m2, iota, _NUM_EXPERTS), axis=1, keepdims=True)
        t = jnp.exp(m2 - m1)
        w1 = 1.0 / (1.0 + t)
        w2 = 1.0 - w1
        bins = jnp.where(iota == i1, w1, 0.0) + jnp.where(iota == i2, w2, 0.0)
        w_ref[...] = jnp.sum(bins, axis=0)[None, :] * (1.0 / _B)

    @pl.when(i >= _B)
    def _combine():
        comb_ref[...] = jnp.dot(w_ref[...], a_ref[...],
                                preferred_element_type=jnp.float32)


@functools.partial(jax.jit, static_argnums=())
def kernel(inputs, W_route, b_route, W_noise, b_noise, A_logs, noise_eps):
    x3 = inputs.reshape(_B, _D_IN, _HW)
    a2 = A_logs.reshape(_NUM_EXPERTS, _DN)
    br = b_route.reshape(1, _NUM_EXPERTS)
    bn = b_noise.reshape(1, _NUM_EXPERTS)

    grid = (_B + _N_COMBINE,)
    comb, z = pl.pallas_call(
        _fused_kernel,
        grid=grid,
        in_specs=[
            pl.BlockSpec((1, _D_IN, _HW), lambda i: (jnp.minimum(i, _B - 1), 0, 0)),
            pl.BlockSpec((_D_IN, _NUM_EXPERTS), lambda i: (0, 0)),
            pl.BlockSpec((1, _NUM_EXPERTS), lambda i: (0, 0)),
            pl.BlockSpec((_D_IN, _NUM_EXPERTS), lambda i: (0, 0)),
            pl.BlockSpec((1, _NUM_EXPERTS), lambda i: (0, 0)),
            pl.BlockSpec((_B, _NUM_EXPERTS), lambda i: (0, 0)),
            pl.BlockSpec((_NUM_EXPERTS, _A_BLK),
                         lambda i: (0, jnp.maximum(i - _B, 0))),
        ],
        out_specs=[
            pl.BlockSpec((1, _A_BLK), lambda i: (0, jnp.maximum(i - _B, 0))),
            pl.BlockSpec(memory_space=pltpu.SMEM),
        ],
        out_shape=[
            jax.ShapeDtypeStruct((1, _DN), jnp.float32),
            jax.ShapeDtypeStruct((1, 1), jnp.float32),
        ],
        scratch_shapes=[
            pltpu.VMEM((_B, _D_IN), jnp.float32),
            pltpu.VMEM((1, _NUM_EXPERTS), jnp.float32),
        ],
    )(x3, W_route, br, W_noise, bn, noise_eps, a2)

    combined = comb.reshape(8192, 64)
    z_loss = z.reshape(())
    return (combined, z_loss)


# speedup vs baseline: 2.1748x; 1.2709x over previous
import functools

import jax
import jax.numpy as jnp
from jax import lax
from jax.experimental import pallas as pl
from jax.experimental.pallas import tpu as pltpu

_NUM_EXPERTS = 16
_TOP_K = 2
_B = 32
_D_IN = 2048
_HW = 256
_D = 8192
_N = 64
_N_COMBINE = 16
_A_BLK = _D // _N_COMBINE


def _fused_kernel(x_ref, wr_ref, br_ref, wn_ref, bn_ref, eps_ref, a_ref,
                  comb_ref, z_ref, mh_ref, w_ref):
    i = pl.program_id(0)

    @pl.when(i < _B)
    def _reduce():
        x = x_ref[0]
        mh_ref[pl.ds(i, 1), :] = jnp.sum(x, axis=-1)[None, :] * (1.0 / _HW)

    @pl.when(i == _B)
    def _route():
        mh = mh_ref[...]
        lin_r = jnp.dot(mh, wr_ref[...], preferred_element_type=jnp.float32) + br_ref[...]
        p = jax.nn.softmax(lin_r, axis=-1)
        lin_n = jnp.dot(mh, wn_ref[...], preferred_element_type=jnp.float32) + bn_ref[...]
        s = eps_ref[...] * jax.nn.softplus(lin_n)
        q = jax.nn.softmax(s, axis=-1)
        noisy = p + q

        zl = jnp.log(jnp.sum(jnp.exp(noisy), axis=-1))
        z_ref[0, 0] = jnp.sum(zl * zl) * (1.0 / _B)

        iota = lax.broadcasted_iota(jnp.int32, (_B, _NUM_EXPERTS), 1)
        m1 = jnp.max(noisy, axis=1, keepdims=True)
        i1 = jnp.min(jnp.where(noisy == m1, iota, _NUM_EXPERTS), axis=1, keepdims=True)
        rest = jnp.where(iota == i1, -jnp.inf, noisy)
        m2 = jnp.max(rest, axis=1, keepdims=True)
        i2 = jnp.min(jnp.where(rest == m2, iota, _NUM_EXPERTS), axis=1, keepdims=True)
        t = jnp.exp(m2 - m1)
        w1 = 1.0 / (1.0 + t)
        w2 = 1.0 - w1
        bins = jnp.where(iota == i1, w1, 0.0) + jnp.where(iota == i2, w2, 0.0)
        binsum = jnp.sum(bins, axis=0, keepdims=True) * (1.0 / _B)
        iota_r = lax.broadcasted_iota(jnp.int32, (1, _NUM_EXPERTS), 1)
        for e in range(_NUM_EXPERTS):
            w_ref[0, e] = jnp.sum(jnp.where(iota_r == e, binsum, 0.0))

    @pl.when(i >= _B)
    def _combine():
        acc = w_ref[0, 0] * a_ref[0]
        for e in range(1, _NUM_EXPERTS):
            acc = acc + w_ref[0, e] * a_ref[e]
        comb_ref[...] = acc


@functools.partial(jax.jit, static_argnums=())
def kernel(inputs, W_route, b_route, W_noise, b_noise, A_logs, noise_eps):
    x3 = inputs.reshape(_B, _D_IN, _HW)
    br = b_route.reshape(1, _NUM_EXPERTS)
    bn = b_noise.reshape(1, _NUM_EXPERTS)

    grid = (_B + _N_COMBINE,)
    comb, z = pl.pallas_call(
        _fused_kernel,
        grid=grid,
        in_specs=[
            pl.BlockSpec((1, _D_IN, _HW), lambda i: (jnp.minimum(i, _B - 1), 0, 0)),
            pl.BlockSpec((_D_IN, _NUM_EXPERTS), lambda i: (0, 0)),
            pl.BlockSpec((1, _NUM_EXPERTS), lambda i: (0, 0)),
            pl.BlockSpec((_D_IN, _NUM_EXPERTS), lambda i: (0, 0)),
            pl.BlockSpec((1, _NUM_EXPERTS), lambda i: (0, 0)),
            pl.BlockSpec((_B, _NUM_EXPERTS), lambda i: (0, 0)),
            pl.BlockSpec((_NUM_EXPERTS, _A_BLK, _N),
                         lambda i: (0, jnp.maximum(i - _B, 0), 0)),
        ],
        out_specs=[
            pl.BlockSpec((_A_BLK, _N), lambda i: (jnp.maximum(i - _B, 0), 0)),
            pl.BlockSpec(memory_space=pltpu.SMEM),
        ],
        out_shape=[
            jax.ShapeDtypeStruct((_D, _N), jnp.float32),
            jax.ShapeDtypeStruct((1, 1), jnp.float32),
        ],
        scratch_shapes=[
            pltpu.VMEM((_B, _D_IN), jnp.float32),
            pltpu.SMEM((1, _NUM_EXPERTS), jnp.float32),
        ],
    )(x3, W_route, br, W_noise, bn, noise_eps, A_logs)

    z_loss = z.reshape(())
    return (comb, z_loss)


# speedup vs baseline: 8.3980x; 3.8615x over previous
import functools

import jax
import jax.numpy as jnp
from jax import lax
from jax.experimental import pallas as pl
from jax.experimental.pallas import tpu as pltpu

_E = 16
_B = 32
_DIN = 2048
_HW = 256
_D = 8192
_N = 64
_NC = 16
_DBLK = _D // _NC


def _fused_kernel(x_ref, wrt_ref, br_ref, wnt_ref, bn_ref, epst_ref, at_ref,
                  combt_ref, z_ref, mh_ref, w_ref):
    i = pl.program_id(0)

    @pl.when(i < _B)
    def _reduce():
        x = x_ref[0]
        mh_ref[pl.ds(i, 1), :] = jnp.sum(x, axis=0)[None, :] * (1.0 / _HW)

    @pl.when(i == _B)
    def _route():
        mh = mh_ref[...]
        dn = (((1,), (1,)), ((), ()))
        iota_c = lax.broadcasted_iota(jnp.int32, (_E, 1), 0)
        br_col = jnp.zeros((_E, 1), jnp.float32)
        bn_col = jnp.zeros((_E, 1), jnp.float32)
        for e in range(_E):
            br_col = jnp.where(iota_c == e, br_ref[0, e], br_col)
            bn_col = jnp.where(iota_c == e, bn_ref[0, e], bn_col)
        lin_r = lax.dot_general(wrt_ref[...], mh, dn,
                                preferred_element_type=jnp.float32) + br_col
        p = jax.nn.softmax(lin_r, axis=0)
        lin_n = lax.dot_general(wnt_ref[...], mh, dn,
                                preferred_element_type=jnp.float32) + bn_col
        s = epst_ref[...] * jax.nn.softplus(lin_n)
        q = jax.nn.softmax(s, axis=0)
        noisy = p + q

        zl = jnp.log(jnp.sum(jnp.exp(noisy), axis=0, keepdims=True))
        z_ref[0, 0] = jnp.sum(zl * zl) * (1.0 / _B)

        iota = lax.broadcasted_iota(jnp.int32, (_E, _B), 0)
        m1 = jnp.max(noisy, axis=0, keepdims=True)
        i1 = jnp.min(jnp.where(noisy == m1, iota, _E), axis=0, keepdims=True)
        rest = jnp.where(iota == i1, -jnp.inf, noisy)
        m2 = jnp.max(rest, axis=0, keepdims=True)
        i2 = jnp.min(jnp.where(rest == m2, iota, _E), axis=0, keepdims=True)
        t = jnp.exp(m2 - m1)
        w1 = 1.0 / (1.0 + t)
        w2 = 1.0 - w1
        bins = jnp.where(iota == i1, w1, 0.0) + jnp.where(iota == i2, w2, 0.0)
        binsum = jnp.sum(bins, axis=1, keepdims=True) * (1.0 / _B)
        for e in range(_E):
            w_ref[0, e] = jnp.sum(jnp.where(iota_c == e, binsum, 0.0))

    @pl.when(i >= _B)
    def _combine():
        acc = w_ref[0, 0] * at_ref[0]
        for e in range(1, _E):
            acc = acc + w_ref[0, e] * at_ref[e]
        combt_ref[...] = acc


@functools.partial(jax.jit, static_argnums=())
def kernel(inputs, W_route, b_route, W_noise, b_noise, A_logs, noise_eps):
    x4 = jnp.transpose(inputs, (0, 2, 3, 1)).reshape(_B, _HW, _DIN)
    wrt = W_route.T
    wnt = W_noise.T
    at = jnp.transpose(A_logs, (0, 2, 1))
    epst = noise_eps.T
    br = b_route.reshape(1, _E)
    bn = b_noise.reshape(1, _E)

    grid = (_B + _NC,)
    combt, z = pl.pallas_call(
        _fused_kernel,
        grid=grid,
        in_specs=[
            pl.BlockSpec((1, _HW, _DIN), lambda i: (jnp.minimum(i, _B - 1), 0, 0)),
            pl.BlockSpec((_E, _DIN), lambda i: (0, 0)),
            pl.BlockSpec(memory_space=pltpu.SMEM),
            pl.BlockSpec((_E, _DIN), lambda i: (0, 0)),
            pl.BlockSpec(memory_space=pltpu.SMEM),
            pl.BlockSpec((_E, _B), lambda i: (0, 0)),
            pl.BlockSpec((_E, _N, _DBLK), lambda i: (0, 0, jnp.maximum(i - _B, 0))),
        ],
        out_specs=[
            pl.BlockSpec((_N, _DBLK), lambda i: (0, jnp.maximum(i - _B, 0))),
            pl.BlockSpec(memory_space=pltpu.SMEM),
        ],
        out_shape=[
            jax.ShapeDtypeStruct((_N, _D), jnp.float32),
            jax.ShapeDtypeStruct((1, 1), jnp.float32),
        ],
        scratch_shapes=[
            pltpu.VMEM((_B, _DIN), jnp.float32),
            pltpu.SMEM((1, _E), jnp.float32),
        ],
    )(x4, wrt, br, wnt, bn, epst, at)

    combined = combt.T
    z_loss = z.reshape(())
    return (combined, z_loss)
